# baseline (speedup 1.0000x reference)
import jax
import jax.numpy as jnp
import numpy as np
from jax import lax
from jax.experimental import pallas as pl
from jax.experimental.pallas import tpu as pltpu
from jax.experimental.pallas import tpu_sc as plsc

H = W = 2048
NW = 32
ROWS_PER_TILE = H // NW
LANES = 16
NVEC = W // LANES
RB = 8
IMAX = jnp.int32(2147483647)
IMIN = jnp.int32(-2147483648)


def _sc_warp_body(img, xsr, ysr, out, xv, yv, ia, ib, ic, idd,
                  va, vb, vc, vd, wav, wbv, wcv, wdv, orow, s0, s1, s2, s3):
    wid = lax.axis_index("s") * 2 + lax.axis_index("c")
    row0 = wid * ROWS_PER_TILE

    def batch_body(b, carry):
        i0 = row0 + b * RB
        pltpu.sync_copy(xsr.at[pl.ds(i0 * W, RB * W)], xv)
        pltpu.sync_copy(ysr.at[pl.ds(i0 * W, RB * W)], yv)

        def row_body(q, carry2):
            rbase = q * W

            def cbody(k, minmax):
              for u in range(2):
                off = k * (2 * LANES) + u * LANES
                sl = pl.ds(rbase + off, LANES)
                csl = pl.ds(off, LANES)
                xsv = xv[sl]
                ysv = yv[sl]
                xcl = jnp.minimum(jnp.maximum(xsv, -4096.0), 4096.0)
                ycl = jnp.minimum(jnp.maximum(ysv, -4096.0), 4096.0)
                xi = xcl.astype(jnp.int32)
                yi = ycl.astype(jnp.int32)
                x0 = jnp.where(xi.astype(jnp.float32) > xcl, xi - 1, xi)
                y0 = jnp.where(yi.astype(jnp.float32) > ycl, yi - 1, yi)
                x0c = jnp.clip(x0, 0, W - 1)
                x1c = jnp.clip(x0 + 1, 0, W - 1)
                y0c = jnp.clip(y0, 0, H - 1)
                y1c = jnp.clip(y0 + 1, 0, H - 1)
                x0f = x0c.astype(jnp.float32)
                x1f = x1c.astype(jnp.float32)
                y0f = y0c.astype(jnp.float32)
                y1f = y1c.astype(jnp.float32)
                dxa = x1f - xsv
                dxb = xsv - x0f
                dya = y1f - ysv
                dyb = ysv - y0f
                wav[csl] = dxa * dya
                wbv[csl] = dxa * dyb
                wcv[csl] = dxb * dya
                wdv[csl] = dxb * dyb
                yb0 = y0c * W
                yb1 = y1c * W
                fa = yb0 + x0c
                fd = yb1 + x1c
                ia[csl] = fa
                ib[csl] = yb1 + x0c
                ic[csl] = yb0 + x1c
                idd[csl] = fd
                mna, mxa, mnd, mxd = minmax
                minmax = (jnp.minimum(mna, fa), jnp.maximum(mxa, fa),
                          jnp.minimum(mnd, fd), jnp.maximum(mxd, fd))
              return minmax

            big = jnp.full((LANES,), IMAX)
            small = jnp.full((LANES,), IMIN)
            mna, mxa, mnd, mxd = lax.fori_loop(
                0, NVEC // 2, cbody, (big, small, big, small))
            m0 = mna[0]
            d0 = mnd[0]
            uniform = (m0 == mxa[0]) & (d0 == mxd[0])
            for j in range(1, LANES):
                uniform = (uniform & (mna[j] == m0) & (mxa[j] == m0)
                           & (mnd[j] == d0) & (mxd[j] == d0))

            @pl.when(uniform)
            def _():
                ca = pltpu.async_copy(img.at[ia.at[pl.ds(0, 8)]],
                                      va.at[pl.ds(0, 8)], s0)
                cb = pltpu.async_copy(img.at[ib.at[pl.ds(0, 8)]],
                                      vb.at[pl.ds(0, 8)], s1)
                cc = pltpu.async_copy(img.at[ic.at[pl.ds(0, 8)]],
                                      vc.at[pl.ds(0, 8)], s2)
                cd = pltpu.async_copy(img.at[idd.at[pl.ds(0, 8)]],
                                      vd.at[pl.ds(0, 8)], s3)
                ca.wait()
                cb.wait()
                cc.wait()
                cd.wait()
                a_s = va[pl.ds(0, LANES)][0]
                b_s = vb[pl.ds(0, LANES)][0]
                c_s = vc[pl.ds(0, LANES)][0]
                d_s = vd[pl.ds(0, LANES)][0]

                def ubody(k, _):
                    csl = pl.ds(k * LANES, LANES)
                    sl = pl.ds(rbase + k * LANES, LANES)
                    acc = ((wav[csl] * a_s + wbv[csl] * b_s)
                           + wcv[csl] * c_s) + wdv[csl] * d_s
                    orow[sl] = acc
                    return 0

                lax.fori_loop(0, NVEC, ubody, 0)

            @pl.when(jnp.logical_not(uniform))
            def _():
                ca = pltpu.async_copy(img.at[ia], va, s0)
                cb = pltpu.async_copy(img.at[ib], vb, s1)
                cc = pltpu.async_copy(img.at[ic], vc, s2)
                cd = pltpu.async_copy(img.at[idd], vd, s3)
                ca.wait()
                cb.wait()
                cc.wait()
                cd.wait()

                def dbody(k, _):
                    csl = pl.ds(k * LANES, LANES)
                    sl = pl.ds(rbase + k * LANES, LANES)
                    acc = ((wav[csl] * va[csl] + wbv[csl] * vb[csl])
                           + wcv[csl] * vc[csl]) + wdv[csl] * vd[csl]
                    orow[sl] = acc
                    return 0

                lax.fori_loop(0, NVEC, dbody, 0)

            return 0

        lax.fori_loop(0, RB, row_body, 0)
        pltpu.sync_copy(orow, out.at[pl.ds(i0 * W, RB * W)])
        return 0

    lax.fori_loop(0, ROWS_PER_TILE // RB, batch_body, 0)


def _make_warp():
    mesh = plsc.VectorSubcoreMesh(core_axis_name="c", subcore_axis_name="s")
    return pl.kernel(
        _sc_warp_body,
        out_type=jax.ShapeDtypeStruct((H * W,), jnp.float32),
        mesh=mesh,
        compiler_params=pltpu.CompilerParams(use_tc_tiling_on_sc=False),
        scratch_types=[
            pltpu.VMEM((RB * W,), jnp.float32),
            pltpu.VMEM((RB * W,), jnp.float32),
            pltpu.VMEM((W,), jnp.int32),
            pltpu.VMEM((W,), jnp.int32),
            pltpu.VMEM((W,), jnp.int32),
            pltpu.VMEM((W,), jnp.int32),
            pltpu.VMEM((W,), jnp.float32),
            pltpu.VMEM((W,), jnp.float32),
            pltpu.VMEM((W,), jnp.float32),
            pltpu.VMEM((W,), jnp.float32),
            pltpu.VMEM((W,), jnp.float32),
            pltpu.VMEM((W,), jnp.float32),
            pltpu.VMEM((W,), jnp.float32),
            pltpu.VMEM((W,), jnp.float32),
            pltpu.VMEM((RB * W,), jnp.float32),
            pltpu.SemaphoreType.DMA,
            pltpu.SemaphoreType.DMA,
            pltpu.SemaphoreType.DMA,
            pltpu.SemaphoreType.DMA,
        ],
    )


def kernel(input_fmap, theta):
    B, C = 1, 1
    out_H, out_W = H, W
    x = jnp.linspace(0.0, out_W - 1.0, out_W) / (out_W - 1.0)
    y = jnp.linspace(0.0, out_H - 1.0, out_H) / (out_H - 1.0)
    xt, yt = jnp.meshgrid(x, y)
    xt = jnp.tile(xt[None, :, :], (B, 1, 1))
    yt = jnp.tile(yt[None, :, :], (B, 1, 1))
    base_grid = jnp.stack([xt, yt], axis=0)
    bg = jnp.transpose(base_grid, (2, 3, 0, 1))
    ones = jnp.ones((out_H, out_W, 1, B), dtype=jnp.float32)
    bg = jnp.concatenate([bg, ones], axis=2)
    th = jnp.squeeze(jnp.reshape(theta, (B, 2, 3, C)))
    M1 = jnp.array([[1.0, np.pi, 0.2], [np.pi, 1.0, 0.2]], dtype=jnp.float32)
    M2 = jnp.array([[0.5, -np.pi / 2, -0.1], [-np.pi / 2, 0.5, -0.1]],
                   dtype=jnp.float32)
    th = th * M1 + M2
    batch_grids = jnp.matmul(th, bg)
    batch_grids = jnp.transpose(batch_grids, (2, 3, 0, 1))
    xs = batch_grids[0] * (W - 1)
    ys = batch_grids[1] * (H - 1)

    img_flat = jnp.reshape(input_fmap, (H * W,))
    warp = _make_warp()
    out = warp(img_flat, jnp.reshape(xs, (H * W,)), jnp.reshape(ys, (H * W,)))
    return jnp.reshape(out, (B, out_H, out_W, C))


# speedup vs baseline: 1.2351x; 1.2351x over previous
import jax
import jax.numpy as jnp
import numpy as np
from jax import lax
from jax.experimental import pallas as pl
from jax.experimental.pallas import tpu as pltpu
from jax.experimental.pallas import tpu_sc as plsc

H = W = 2048
NW = 32
ROWS_PER_TILE = H // NW
LANES = 16
NVEC = W // LANES
IMAX = jnp.int32(2147483647)
IMIN = jnp.int32(-2147483648)


def _sc_warp_body(img, xsr, ysr, out, xv, yv, ia, ib, ic, idd,
                  va, vb, vc, vd, wav, wbv, wcv, wdv, orow, s0, s1, s2, s3):
    wid = lax.axis_index("s") * 2 + lax.axis_index("c")
    row0 = wid * ROWS_PER_TILE

    def row_body(r, carry):
        i = row0 + r
        pltpu.sync_copy(xsr.at[pl.ds(i * W, W)], xv)
        pltpu.sync_copy(ysr.at[pl.ds(i * W, W)], yv)

        def cbody(k, minmax):
          for u in range(2):
            off = k * (2 * LANES) + u * LANES
            sl = pl.ds(off, LANES)
            xsv = xv[sl]
            ysv = yv[sl]
            xcl = jnp.minimum(jnp.maximum(xsv, -4096.0), 4096.0)
            ycl = jnp.minimum(jnp.maximum(ysv, -4096.0), 4096.0)
            xi = xcl.astype(jnp.int32)
            yi = ycl.astype(jnp.int32)
            x0 = jnp.where(xi.astype(jnp.float32) > xcl, xi - 1, xi)
            y0 = jnp.where(yi.astype(jnp.float32) > ycl, yi - 1, yi)
            x0c = jnp.clip(x0, 0, W - 1)
            x1c = jnp.clip(x0 + 1, 0, W - 1)
            y0c = jnp.clip(y0, 0, H - 1)
            y1c = jnp.clip(y0 + 1, 0, H - 1)
            x0f = x0c.astype(jnp.float32)
            x1f = x1c.astype(jnp.float32)
            y0f = y0c.astype(jnp.float32)
            y1f = y1c.astype(jnp.float32)
            dxa = x1f - xsv
            dxb = xsv - x0f
            dya = y1f - ysv
            dyb = ysv - y0f
            wav[sl] = dxa * dya
            wbv[sl] = dxa * dyb
            wcv[sl] = dxb * dya
            wdv[sl] = dxb * dyb
            yb0 = y0c * W
            yb1 = y1c * W
            fa = yb0 + x0c
            fd = yb1 + x1c
            ia[sl] = fa
            ib[sl] = yb1 + x0c
            ic[sl] = yb0 + x1c
            idd[sl] = fd
            mna, mxa, mnd, mxd = minmax
            minmax = (jnp.minimum(mna, fa), jnp.maximum(mxa, fa),
                      jnp.minimum(mnd, fd), jnp.maximum(mxd, fd))
          return minmax

        big = jnp.full((LANES,), IMAX)
        small = jnp.full((LANES,), IMIN)
        mna, mxa, mnd, mxd = lax.fori_loop(
            0, NVEC // 2, cbody, (big, small, big, small))
        m0 = mna[0]
        d0 = mnd[0]
        uniform = (m0 == mxa[0]) & (d0 == mxd[0])
        for j in range(1, LANES):
            uniform = (uniform & (mna[j] == m0) & (mxa[j] == m0)
                       & (mnd[j] == d0) & (mxd[j] == d0))

        @pl.when(uniform)
        def _():
            fb_s = ib[pl.ds(0, LANES)][0]
            fc_s = ic[pl.ds(0, LANES)][0]
            lane = lax.iota(jnp.int32, LANES)
            l4 = lane & 3
            fvec = jnp.where(l4 == 0, m0,
                             jnp.where(l4 == 1, fb_s,
                                       jnp.where(l4 == 2, fc_s, d0)))
            ia[pl.ds(0, LANES)] = fvec
            pltpu.async_copy(img.at[ia.at[pl.ds(0, 8)]],
                             va.at[pl.ds(0, 8)], s0).wait()
            v8 = va[pl.ds(0, LANES)]
            a_s = v8[0]
            b_s = v8[1]
            c_s = v8[2]
            d_s = v8[3]

            def ubody(k, _):
                sl = pl.ds(k * LANES, LANES)
                acc = ((wav[sl] * a_s + wbv[sl] * b_s)
                       + wcv[sl] * c_s) + wdv[sl] * d_s
                orow[sl] = acc
                return 0

            lax.fori_loop(0, NVEC, ubody, 0)

        @pl.when(jnp.logical_not(uniform))
        def _():
            ca = pltpu.async_copy(img.at[ia], va, s0)
            cb = pltpu.async_copy(img.at[ib], vb, s1)
            cc = pltpu.async_copy(img.at[ic], vc, s2)
            cd = pltpu.async_copy(img.at[idd], vd, s3)
            ca.wait()
            cb.wait()
            cc.wait()
            cd.wait()

            def dbody(k, _):
                sl = pl.ds(k * LANES, LANES)
                acc = ((wav[sl] * va[sl] + wbv[sl] * vb[sl])
                       + wcv[sl] * vc[sl]) + wdv[sl] * vd[sl]
                orow[sl] = acc
                return 0

            lax.fori_loop(0, NVEC, dbody, 0)

        pltpu.sync_copy(orow, out.at[pl.ds(i * W, W)])
        return 0

    lax.fori_loop(0, ROWS_PER_TILE, row_body, 0)


def _make_warp():
    mesh = plsc.VectorSubcoreMesh(core_axis_name="c", subcore_axis_name="s")
    return pl.kernel(
        _sc_warp_body,
        out_type=jax.ShapeDtypeStruct((H * W,), jnp.float32),
        mesh=mesh,
        compiler_params=pltpu.CompilerParams(use_tc_tiling_on_sc=False),
        scratch_types=[
            pltpu.VMEM((W,), jnp.float32),
            pltpu.VMEM((W,), jnp.float32),
            pltpu.VMEM((W,), jnp.int32),
            pltpu.VMEM((W,), jnp.int32),
            pltpu.VMEM((W,), jnp.int32),
            pltpu.VMEM((W,), jnp.int32),
            pltpu.VMEM((W,), jnp.float32),
            pltpu.VMEM((W,), jnp.float32),
            pltpu.VMEM((W,), jnp.float32),
            pltpu.VMEM((W,), jnp.float32),
            pltpu.VMEM((W,), jnp.float32),
            pltpu.VMEM((W,), jnp.float32),
            pltpu.VMEM((W,), jnp.float32),
            pltpu.VMEM((W,), jnp.float32),
            pltpu.VMEM((W,), jnp.float32),
            pltpu.SemaphoreType.DMA,
            pltpu.SemaphoreType.DMA,
            pltpu.SemaphoreType.DMA,
            pltpu.SemaphoreType.DMA,
        ],
    )


def kernel(input_fmap, theta):
    B, C = 1, 1
    out_H, out_W = H, W
    x = jnp.linspace(0.0, out_W - 1.0, out_W) / (out_W - 1.0)
    y = jnp.linspace(0.0, out_H - 1.0, out_H) / (out_H - 1.0)
    xt, yt = jnp.meshgrid(x, y)
    xt = jnp.tile(xt[None, :, :], (B, 1, 1))
    yt = jnp.tile(yt[None, :, :], (B, 1, 1))
    base_grid = jnp.stack([xt, yt], axis=0)
    bg = jnp.transpose(base_grid, (2, 3, 0, 1))
    ones = jnp.ones((out_H, out_W, 1, B), dtype=jnp.float32)
    bg = jnp.concatenate([bg, ones], axis=2)
    th = jnp.squeeze(jnp.reshape(theta, (B, 2, 3, C)))
    M1 = jnp.array([[1.0, np.pi, 0.2], [np.pi, 1.0, 0.2]], dtype=jnp.float32)
    M2 = jnp.array([[0.5, -np.pi / 2, -0.1], [-np.pi / 2, 0.5, -0.1]],
                   dtype=jnp.float32)
    th = th * M1 + M2
    batch_grids = jnp.matmul(th, bg)
    batch_grids = jnp.transpose(batch_grids, (2, 3, 0, 1))
    xs = batch_grids[0] * (W - 1)
    ys = batch_grids[1] * (H - 1)

    img_flat = jnp.reshape(input_fmap, (H * W,))
    warp = _make_warp()
    out = warp(img_flat, jnp.reshape(xs, (H * W,)), jnp.reshape(ys, (H * W,)))
    return jnp.reshape(out, (B, out_H, out_W, C))
